# baseline (speedup 1.0000x reference)
import jax
import jax.numpy as jnp
from jax import lax
from jax.experimental import pallas as pl
from jax.experimental.pallas import tpu as pltpu
from jax.experimental.pallas import tpu_sc as plsc

NUM_CORES = 2
NUM_SUBCORES = 16
LANES = 16
NW = NUM_CORES * NUM_SUBCORES

NUM_ROWS = 1000000
BATCH = 16384
DIM = 64
B_PER_W = BATCH // NW
CHUNK = 128
N_CHUNKS = B_PER_W // CHUNK
GROUPS = CHUNK // LANES


def _sc_body(users_hbm, items_hbm, ut_hbm, it_hbm, out_hbm,
             uidx_v, iidx_v, upair_v, ipair_v, urows_v, irows_v, scores_v,
             usem, isem):
    wid = lax.axis_index("s") * NUM_CORES + lax.axis_index("c")
    base_chunk = wid * N_CHUNKS

    pltpu.sync_copy(users_hbm.at[pl.ds(base_chunk, N_CHUNKS)], uidx_v)
    pltpu.sync_copy(items_hbm.at[pl.ds(base_chunk, N_CHUNKS)], iidx_v)

    def pair_body(q, _):
        cq, gq = q // GROUPS, (q % GROUPS) * LANES
        upair_v[cq, pl.ds(gq, LANES)] = lax.shift_right_logical(
            uidx_v[cq, pl.ds(gq, LANES)], 1)
        ipair_v[cq, pl.ds(gq, LANES)] = lax.shift_right_logical(
            iidx_v[cq, pl.ds(gq, LANES)], 1)
        return 0

    lax.fori_loop(0, N_CHUNKS * GROUPS, pair_body, 0, unroll=4)

    lane_iota = lax.iota(jnp.int32, LANES)

    for c in range(N_CHUNKS):
        ucp = pltpu.async_copy(ut_hbm.at[upair_v.at[c]], urows_v, usem)
        icp = pltpu.async_copy(it_hbm.at[ipair_v.at[c]], irows_v, isem)
        ucp.wait()
        icp.wait()

        def group_body(g, _):
            base = g * LANES
            cols = [jnp.zeros((LANES,), jnp.float32) for _ in range(4)]
            for j in range(LANES):
                prods = [jnp.zeros((LANES,), jnp.float32) for _ in range(4)]
                for k in range(DIM // LANES):
                    u0 = urows_v[base + j, pl.ds(k * LANES, LANES)]
                    u1 = urows_v[base + j, pl.ds(DIM + k * LANES, LANES)]
                    v0 = irows_v[base + j, pl.ds(k * LANES, LANES)]
                    v1 = irows_v[base + j, pl.ds(DIM + k * LANES, LANES)]
                    prods[0] = prods[0] + u0 * v0
                    prods[1] = prods[1] + u0 * v1
                    prods[2] = prods[2] + u1 * v0
                    prods[3] = prods[3] + u1 * v1
                lane_j = lane_iota == j
                for t in range(4):
                    cols[t] = jnp.where(lane_j, jnp.sum(prods[t]), cols[t])
            upar = (uidx_v[c, pl.ds(base, LANES)] & 1) == 1
            ipar = (iidx_v[c, pl.ds(base, LANES)] & 1) == 1
            col = jnp.where(
                upar,
                jnp.where(ipar, cols[3], cols[2]),
                jnp.where(ipar, cols[1], cols[0]),
            )
            score = 1.0 / (1.0 + jnp.exp(-col))
            scores_v[pl.ds(c * CHUNK + g * LANES, LANES)] = score
            return 0

        lax.fori_loop(0, GROUPS, group_body, 0)

    pltpu.sync_copy(scores_v, out_hbm.at[pl.ds(wid * B_PER_W, B_PER_W)])


@jax.jit
def kernel(users, items, user_table, item_table):
    users2 = users.reshape(BATCH // CHUNK, CHUNK)
    items2 = items.reshape(BATCH // CHUNK, CHUNK)
    ut2 = user_table.reshape(NUM_ROWS // 2, 2 * DIM)
    it2 = item_table.reshape(NUM_ROWS // 2, 2 * DIM)
    mesh = plsc.VectorSubcoreMesh(core_axis_name="c", subcore_axis_name="s")
    run = pl.kernel(
        _sc_body,
        out_type=jax.ShapeDtypeStruct((BATCH,), jnp.float32),
        mesh=mesh,
        scratch_types=[
            pltpu.VMEM((N_CHUNKS, CHUNK), jnp.int32),
            pltpu.VMEM((N_CHUNKS, CHUNK), jnp.int32),
            pltpu.VMEM((N_CHUNKS, CHUNK), jnp.int32),
            pltpu.VMEM((N_CHUNKS, CHUNK), jnp.int32),
            pltpu.VMEM((CHUNK, 2 * DIM), jnp.float32),
            pltpu.VMEM((CHUNK, 2 * DIM), jnp.float32),
            pltpu.VMEM((B_PER_W,), jnp.float32),
            pltpu.SemaphoreType.DMA,
            pltpu.SemaphoreType.DMA,
        ],
        compiler_params=pltpu.CompilerParams(
            needs_layout_passes=False, use_tc_tiling_on_sc=True),
    )
    return run(users2, items2, ut2, it2)
